# baseline (speedup 1.0000x reference)
import jax
import jax.numpy as jnp
from jax.experimental import pallas as pl

_K = 16
_R = 128


def _voroloss_block(sp_ref, pT_ref, out_ref):
    S = sp_ref[...]
    P = pT_ref[...]
    M = S.shape[0]
    R = P.shape[1]

    sx = S[:, 0:1]
    sy = S[:, 1:2]
    sz = S[:, 2:3]
    px = P[0:1, :]
    py = P[1:2, :]
    pz = P[2:3, :]

    f32 = jnp.float32
    s2 = (sx * sx + sy * sy) + sz * sz
    p2 = (px * px + py * py) + pz * pz

    bf = jnp.bfloat16
    sxb = sx.astype(bf).astype(f32)
    syb = sy.astype(bf).astype(f32)
    szb = sz.astype(bf).astype(f32)
    pxb = px.astype(bf).astype(f32)
    pyb = py.astype(bf).astype(f32)
    pzb = pz.astype(bf).astype(f32)
    ps = (sxb * pxb + syb * pyb) + szb * pzb
    d2 = (p2 + s2) - 2.0 * ps

    inf = f32(jnp.inf)
    m0 = jnp.min(d2, axis=0, keepdims=True)
    m = m0
    for _ in range(_K - 1):
        m = jnp.min(jnp.where(d2 > m, d2, inf), axis=0, keepdims=True)
    T = m

    ii = jax.lax.broadcasted_iota(jnp.int32, (M, R), 0)
    i0 = jnp.min(jnp.where(d2 == m0, ii, M), axis=0, keepdims=True)
    sel0 = ii == i0
    zero = f32(0.0)
    s0x = jnp.sum(jnp.where(sel0, sx, zero), axis=0, keepdims=True)
    s0y = jnp.sum(jnp.where(sel0, sy, zero), axis=0, keepdims=True)
    s0z = jnp.sum(jnp.where(sel0, sz, zero), axis=0, keepdims=True)

    g = s2 - 2.0 * ((sx * px + sy * py) + sz * pz)
    g0 = jnp.sum(jnp.where(sel0, g, zero), axis=0, keepdims=True)
    ex = sx - s0x
    ey = sy - s0y
    ez = sz - s0z
    el2 = ex * ex + ey * ey + ez * ez
    diff = g - g0
    f = (diff * diff) / (4.0 * el2)
    mask = (d2 <= T) & jnp.logical_not(sel0)
    res = jnp.min(jnp.where(mask, f, inf), axis=0, keepdims=True)
    out_ref[...] = res[None]


@jax.jit
def kernel(points, spoints):
    N = points.shape[0]
    M = spoints.shape[0]
    R = _R
    grid = N // R
    pT = points.T
    out = pl.pallas_call(
        _voroloss_block,
        grid=(grid,),
        in_specs=[
            pl.BlockSpec((M, 3), lambda i: (0, 0)),
            pl.BlockSpec((3, R), lambda i: (0, i)),
        ],
        out_specs=pl.BlockSpec((1, 1, R), lambda i: (i, 0, 0)),
        out_shape=jax.ShapeDtypeStruct((grid, 1, R), jnp.float32),
    )(spoints, pT)
    return out.reshape(N)


# speedup vs baseline: 1.7693x; 1.7693x over previous
import jax
import jax.numpy as jnp
from jax.experimental import pallas as pl

_K = 16
_R = 256
_CH = 512

_DN = (((1,), (0,)), ((), ()))


def _chunk_min(x, M, R):
    parts = [
        jnp.min(x[c:c + _CH, :], axis=0, keepdims=True)
        for c in range(0, M, _CH)
    ]
    acc = parts[0]
    for p_ in parts[1:]:
        acc = jnp.minimum(acc, p_)
    return acc


def _masked_chunk_min(d2, m, inf, M, R):
    parts = []
    for c in range(0, M, _CH):
        blk = d2[c:c + _CH, :]
        parts.append(
            jnp.min(jnp.where(blk > m, blk, inf), axis=0, keepdims=True))
    acc = parts[0]
    for p_ in parts[1:]:
        acc = jnp.minimum(acc, p_)
    return acc


def _voroloss_block(sp_ref, pT_ref, out_ref):
    S = sp_ref[...]
    P = pT_ref[...]
    M = S.shape[0]
    R = P.shape[1]

    sx = S[:, 0:1]
    sy = S[:, 1:2]
    sz = S[:, 2:3]
    px = P[0:1, :]
    py = P[1:2, :]
    pz = P[2:3, :]

    f32 = jnp.float32
    bf = jnp.bfloat16
    s2 = (sx * sx + sy * sy) + sz * sz
    p2 = (px * px + py * py) + pz * pz

    ps = jax.lax.dot_general(S.astype(bf), P.astype(bf), _DN,
                             preferred_element_type=f32)
    d2 = (p2 + s2) - 2.0 * ps

    inf = f32(jnp.inf)
    m0 = _chunk_min(d2, M, R)
    m = m0
    for _ in range(_K - 1):
        m = _masked_chunk_min(d2, m, inf, M, R)
    T = m

    ic = jax.lax.broadcasted_iota(jnp.int32, (M, 1), 0)
    i0 = jnp.min(jnp.where(d2 == m0, ic, M), axis=0, keepdims=True)
    sel0 = ic == i0
    zero = f32(0.0)
    s0x = jnp.sum(jnp.where(sel0, sx, zero), axis=0, keepdims=True)
    s0y = jnp.sum(jnp.where(sel0, sy, zero), axis=0, keepdims=True)
    s0z = jnp.sum(jnp.where(sel0, sz, zero), axis=0, keepdims=True)
    S0 = jnp.concatenate([s0x, s0y, s0z], axis=0)

    hi = jax.lax.Precision.HIGHEST
    G = jax.lax.dot_general(S, P, _DN, precision=hi,
                            preferred_element_type=f32)
    g = s2 - 2.0 * G
    s02 = (s0x * s0x + s0y * s0y) + s0z * s0z
    g0 = s02 - 2.0 * ((px * s0x + py * s0y) + pz * s0z)
    E = jax.lax.dot_general(S, S0, _DN, precision=hi,
                            preferred_element_type=f32)
    el2 = (s2 + s02) - 2.0 * E
    diff = g - g0
    q = (diff * diff) / el2
    fmask = (d2 <= T) & jnp.logical_not(sel0)
    res = jnp.min(jnp.where(fmask, q, inf), axis=0, keepdims=True)
    out_ref[...] = (0.25 * res)[None]


@jax.jit
def kernel(points, spoints):
    N = points.shape[0]
    M = spoints.shape[0]
    R = _R
    grid = N // R
    pT = points.T
    out = pl.pallas_call(
        _voroloss_block,
        grid=(grid,),
        in_specs=[
            pl.BlockSpec((M, 3), lambda i: (0, 0)),
            pl.BlockSpec((3, R), lambda i: (0, i)),
        ],
        out_specs=pl.BlockSpec((1, 1, R), lambda i: (i, 0, 0)),
        out_shape=jax.ShapeDtypeStruct((grid, 1, R), jnp.float32),
    )(spoints, pT)
    return out.reshape(N)


# speedup vs baseline: 1.7811x; 1.0067x over previous
import jax
import jax.numpy as jnp
from jax.experimental import pallas as pl

_K = 16
_R = 256
_H = 16

_DN = (((1,), (0,)), ((), ()))


def _sort16(u):
    u = list(u)
    k = 2
    while k <= 16:
        j = k // 2
        while j >= 1:
            for i in range(16):
                p = i ^ j
                if p > i:
                    lo = jnp.minimum(u[i], u[p])
                    hi = jnp.maximum(u[i], u[p])
                    if (i & k) == 0:
                        u[i], u[p] = lo, hi
                    else:
                        u[i], u[p] = hi, lo
            j //= 2
        k *= 2
    return u


def _resort16(u):
    u = list(u)
    j = 8
    while j >= 1:
        for i in range(16):
            p = i ^ j
            if p > i:
                lo = jnp.minimum(u[i], u[p])
                hi = jnp.maximum(u[i], u[p])
                u[i], u[p] = lo, hi
        j //= 2
    return u


def _merge_lo(a, b):
    c = [jnp.minimum(a[i], b[15 - i]) for i in range(16)]
    return _resort16(c)


def _top16_rows(d2, M):
    units = [d2[i * _H:(i + 1) * _H, :] for i in range(M // _H)]
    runs = [_sort16(units[16 * r:16 * (r + 1)])
            for r in range(len(units) // 16)]
    while len(runs) > 1:
        runs = [_merge_lo(runs[2 * i], runs[2 * i + 1])
                for i in range(len(runs) // 2)]
    return jnp.concatenate(runs[0], axis=0)


def _voroloss_block(sp_ref, pT_ref, out_ref):
    S = sp_ref[...]
    P = pT_ref[...]
    M = S.shape[0]
    R = P.shape[1]

    sx = S[:, 0:1]
    sy = S[:, 1:2]
    sz = S[:, 2:3]
    px = P[0:1, :]
    py = P[1:2, :]
    pz = P[2:3, :]

    f32 = jnp.float32
    bf = jnp.bfloat16
    s2 = (sx * sx + sy * sy) + sz * sz
    p2 = (px * px + py * py) + pz * pz

    ps = jax.lax.dot_general(S.astype(bf), P.astype(bf), _DN,
                             preferred_element_type=f32)
    d2 = (p2 + s2) - 2.0 * ps

    inf = f32(jnp.inf)
    cand = _top16_rows(d2, M)
    m0 = jnp.min(cand, axis=0, keepdims=True)
    m = m0
    for _ in range(_K - 1):
        m = jnp.min(jnp.where(cand > m, cand, inf),
                    axis=0, keepdims=True)
    T = m

    ic = jax.lax.broadcasted_iota(jnp.int32, (M, 1), 0)
    i0 = jnp.min(jnp.where(d2 == m0, ic, M), axis=0, keepdims=True)
    sel0 = ic == i0
    zero = f32(0.0)
    s0x = jnp.sum(jnp.where(sel0, sx, zero), axis=0, keepdims=True)
    s0y = jnp.sum(jnp.where(sel0, sy, zero), axis=0, keepdims=True)
    s0z = jnp.sum(jnp.where(sel0, sz, zero), axis=0, keepdims=True)
    S0 = jnp.concatenate([s0x, s0y, s0z], axis=0)

    hi = jax.lax.Precision.HIGHEST
    G = jax.lax.dot_general(S, P, _DN, precision=hi,
                            preferred_element_type=f32)
    g = s2 - 2.0 * G
    s02 = (s0x * s0x + s0y * s0y) + s0z * s0z
    g0 = s02 - 2.0 * ((px * s0x + py * s0y) + pz * s0z)
    E = jax.lax.dot_general(S, S0, _DN, precision=hi,
                            preferred_element_type=f32)
    el2 = (s2 + s02) - 2.0 * E
    diff = g - g0
    q = (diff * diff) / el2
    fmask = (d2 <= T) & jnp.logical_not(sel0)
    res = jnp.min(jnp.where(fmask, q, inf), axis=0, keepdims=True)
    out_ref[...] = (0.25 * res)[None]


@jax.jit
def kernel(points, spoints):
    N = points.shape[0]
    M = spoints.shape[0]
    R = _R
    grid = N // R
    pT = points.T
    out = pl.pallas_call(
        _voroloss_block,
        grid=(grid,),
        in_specs=[
            pl.BlockSpec((M, 3), lambda i: (0, 0)),
            pl.BlockSpec((3, R), lambda i: (0, i)),
        ],
        out_specs=pl.BlockSpec((1, 1, R), lambda i: (i, 0, 0)),
        out_shape=jax.ShapeDtypeStruct((grid, 1, R), jnp.float32),
    )(spoints, pT)
    return out.reshape(N)
